# Initial kernel scaffold; baseline (speedup 1.0000x reference)
import functools

import jax
import jax.numpy as jnp
from jax import lax
from jax.experimental import pallas as pl
from jax.experimental.pallas import tpu as pltpu
from jax.experimental.pallas import tpu_sc as plsc

B = 16384
F = 26
V = 100000
BASE_DIM = 128
NUM_GROUPS = 4
BLOCK_DIM = 32
RPG = 650000

NC, NS = 2, 16
NW = NC * NS
ROWS_PER_W = B // NW
CB = 64
NSUB = ROWS_PER_W // CB
NCH = CB // 16

_SLOTS = []
for _g, _fs in ((0, range(0, 6)), (1, range(7, 13)),
                (2, range(13, 19)), (3, range(20, 26))):
    for _j, _f in enumerate(_fs):
        _SLOTS.append((_g, _j, _f, _f * V - _g * RPG))
_EXTRAS = [
    (24, 0, 6, True),
    (25, 1, 6, False),
    (26, 2, 19, True),
    (27, 3, 19, False),
]


def _sc_body(x_hbm, t0, t1, t2, t3, out_hbm, xv, idxb, wbuf, rows, pooled, sem):
    tables = (t0, t1, t2, t3)
    wid = lax.axis_index("s") * NC + lax.axis_index("c")
    base_row = wid * ROWS_PER_W

    pltpu.sync_copy(x_hbm.at[pl.ds(base_row * F, ROWS_PER_W * F)], xv)

    lanes = lax.iota(jnp.int32, 16)

    def sub_chunk(sub, _):
        sub_base = sub * CB

        for c in range(NCH):
            gidx = lanes * F + (sub_base + c * 16) * F
            xf = {}
            for f in sorted({s[2] for s in _SLOTS} | {6, 19}):
                xf[f] = plsc.load_gather(xv, [gidx + f])
            for g, j, f, ofs in _SLOTS:
                idxb[g * 6 + j, pl.ds(c * 16, 16)] = xf[f] + ofs
            for slot, tbl, f, below in _EXTRAS:
                xc = xf[f]
                if below:
                    idxv = jnp.minimum(xc + 600000, RPG - 1)
                    wv = jnp.where(xc < 50000, 1.0, 0.0)
                else:
                    idxv = jnp.maximum(xc - 50000, 0)
                    wv = jnp.where(xc >= 50000, 1.0, 0.0)
                idxb[slot, pl.ds(c * 16, 16)] = idxv
                wbuf[slot - 24, pl.ds(c * 16, 16)] = wv.astype(jnp.float32)

        descs = []
        for g, j, f, ofs in _SLOTS:
            k = g * 6 + j
            descs.append(
                pltpu.async_copy(tables[g].at[idxb.at[k]], rows.at[k], sem))
        for slot, tbl, f, below in _EXTRAS:
            descs.append(
                pltpu.async_copy(tables[tbl].at[idxb.at[slot]], rows.at[slot],
                                 sem))
        for d in descs:
            d.wait()

        def accum(r, _):
            ridx = jnp.full((16,), r, jnp.int32)
            for g in range(NUM_GROUPS):
                wv = plsc.load_gather(wbuf, [jnp.full((16,), g, jnp.int32),
                                             ridx])
                for dh in (0, 16):
                    acc = rows[g * 6, r, pl.ds(dh, 16)]
                    for j in range(1, 6):
                        acc = acc + rows[g * 6 + j, r, pl.ds(dh, 16)]
                    acc = acc + wv * rows[24 + g, r, pl.ds(dh, 16)]
                    pooled[r, pl.ds(g * 32 + dh, 16)] = acc
            return 0

        lax.fori_loop(0, CB, accum, 0)

        pltpu.sync_copy(pooled,
                        out_hbm.at[pl.ds(base_row + sub_base, CB), :])
        return 0

    lax.fori_loop(0, NSUB, sub_chunk, 0)


@functools.partial(
    pl.kernel,
    out_type=jax.ShapeDtypeStruct((B, BASE_DIM), jnp.float32),
    mesh=plsc.VectorSubcoreMesh(core_axis_name="c", subcore_axis_name="s"),
    scratch_types=[
        pltpu.VMEM((ROWS_PER_W * F,), jnp.int32),
        pltpu.VMEM((28, CB), jnp.int32),
        pltpu.VMEM((4, CB), jnp.float32),
        pltpu.VMEM((28, CB, BLOCK_DIM), jnp.float32),
        pltpu.VMEM((CB, BASE_DIM), jnp.float32),
        pltpu.SemaphoreType.DMA,
    ],
)
def _sc_pool(x_hbm, t0, t1, t2, t3, out_hbm, xv, idxb, wbuf, rows, pooled,
             sem):
    _sc_body(x_hbm, t0, t1, t2, t3, out_hbm, xv, idxb, wbuf, rows, pooled,
             sem)


_BM = 1024


def _mm_body(p_ref, l0, l1, l2, l3, o_ref):
    w = jnp.concatenate([l0[...].T, l1[...].T, l2[...].T, l3[...].T], axis=0)
    o_ref[...] = jnp.dot(p_ref[...], w, preferred_element_type=jnp.float32)


def _project(pooled, l0, l1, l2, l3):
    wspec = pl.BlockSpec((BASE_DIM, BLOCK_DIM), lambda i: (0, 0))
    return pl.pallas_call(
        _mm_body,
        grid=(B // _BM,),
        in_specs=[pl.BlockSpec((_BM, BASE_DIM), lambda i: (i, 0)),
                  wspec, wspec, wspec, wspec],
        out_specs=pl.BlockSpec((_BM, BASE_DIM), lambda i: (i, 0)),
        out_shape=jax.ShapeDtypeStruct((B, BASE_DIM), jnp.float32),
    )(pooled, l0, l1, l2, l3)


def kernel(x, embed_w0, embed_w1, embed_w2, embed_w3,
           linear_w0, linear_w1, linear_w2, linear_w3):
    x_flat = x.astype(jnp.int32).reshape(-1)
    pooled = _sc_pool(x_flat, embed_w0, embed_w1, embed_w2, embed_w3)
    return _project(pooled, linear_w0, linear_w1, linear_w2, linear_w3)


# speedup vs baseline: 7.6776x; 7.6776x over previous
import functools

import jax
import jax.numpy as jnp
from jax import lax
from jax.experimental import pallas as pl
from jax.experimental.pallas import tpu as pltpu
from jax.experimental.pallas import tpu_sc as plsc

B = 16384
F = 26
V = 100000
BASE_DIM = 128
NUM_GROUPS = 4
BLOCK_DIM = 32
RPG = 650000
CHW = 2048
NCHUNK = (RPG + CHW - 1) // CHW
LINES = NCHUNK * (CHW // 4)

NC, NS = 2, 16
NW = NC * NS
ROWS_PER_W = B // NW
CB = 32
NSUB = ROWS_PER_W // CB
NCH = CB // 16

_GROUP_FIELDS = (range(0, 6), range(7, 13), range(13, 19), range(20, 26))
_GROUP_EXTRA = ((6, True), (6, False), (19, True), (19, False))


def _repack_body(i0, i1, i2, i3, o0, o1, o2, o3):
    for i_ref, o_ref in ((i0, o0), (i1, o1), (i2, o2), (i3, o3)):
        x = i_ref[...]
        for p in range(CHW // 512):
            o_ref[p * 128:(p + 1) * 128, :] = jnp.concatenate(
                [x[:, p * 512 + a * 128:p * 512 + (a + 1) * 128].T
                 for a in range(4)], axis=1)


def _repack(t0, t1, t2, t3):
    ispec = pl.BlockSpec((32, CHW), lambda i: (0, i))
    ospec = pl.BlockSpec((CHW // 4, 128), lambda i: (i, 0))
    oshape = jax.ShapeDtypeStruct((LINES, 128), jnp.float32)
    return pl.pallas_call(
        _repack_body,
        grid=(NCHUNK,),
        in_specs=[ispec] * 4,
        out_specs=[ospec] * 4,
        out_shape=[oshape] * 4,
    )(t0, t1, t2, t3)


def _line_off(local):
    line = (local >> 9) * 128 + (local & 127)
    aoff = ((local >> 7) & 3) * 32
    return line, aoff


def _sc_body(x_hbm, t0, t1, t2, t3, out_hbm, xv, idxb, offb, wbuf, rows,
             pooled, sem):
    tables = (t0, t1, t2, t3)
    wid = lax.axis_index("s") * NC + lax.axis_index("c")
    base_row = wid * ROWS_PER_W

    pltpu.sync_copy(x_hbm.at[pl.ds(base_row * F, ROWS_PER_W * F)], xv)

    lanes = lax.iota(jnp.int32, 16)

    def sub_chunk(sub, _):
        sub_base = sub * CB

        for gp in range(2):
            for c in range(NCH):
                gidx = lanes * F + (sub_base + c * 16) * F
                xf = {}
                for gi in range(2):
                    g = 2 * gp + gi
                    for j, f in enumerate(_GROUP_FIELDS[g]):
                        if f not in xf:
                            xf[f] = plsc.load_gather(xv, [gidx + f])
                        k = gi * 7 + j
                        local = xf[f] + (f * V - g * RPG)
                        line, aoff = _line_off(local)
                        idxb[k, pl.ds(c * 16, 16)] = line
                        offb[k, pl.ds(c * 16, 16)] = aoff
                    ef, below = _GROUP_EXTRA[g]
                    if ef not in xf:
                        xf[ef] = plsc.load_gather(xv, [gidx + ef])
                    xc = xf[ef]
                    if below:
                        local = jnp.minimum(xc + 600000, RPG - 1)
                        wv = jnp.where(xc < 50000, 1.0, 0.0)
                    else:
                        local = jnp.maximum(xc - 50000, 0)
                        wv = jnp.where(xc >= 50000, 1.0, 0.0)
                    k = gi * 7 + 6
                    line, aoff = _line_off(local)
                    idxb[k, pl.ds(c * 16, 16)] = line
                    offb[k, pl.ds(c * 16, 16)] = aoff
                    wbuf[gi, pl.ds(c * 16, 16)] = wv.astype(jnp.float32)

            descs = []
            for gi in range(2):
                g = 2 * gp + gi
                for j in range(7):
                    k = gi * 7 + j
                    descs.append(pltpu.async_copy(
                        tables[g].at[idxb.at[k]], rows.at[k], sem))
            for d in descs:
                d.wait()

            def accum(r, _):
                ridx = jnp.full((16,), r, jnp.int32)
                for gi in range(2):
                    g = 2 * gp + gi
                    wv = plsc.load_gather(
                        wbuf, [jnp.full((16,), gi, jnp.int32), ridx])
                    kv = [jnp.full((16,), gi * 7 + j, jnp.int32)
                          for j in range(7)]
                    offs = [plsc.load_gather(offb, [kv[j], ridx])
                            for j in range(7)]
                    for dh in (0, 16):
                        acc = plsc.load_gather(
                            rows, [kv[0], ridx, offs[0] + dh + lanes])
                        for j in range(1, 6):
                            acc = acc + plsc.load_gather(
                                rows, [kv[j], ridx, offs[j] + dh + lanes])
                        acc = acc + wv * plsc.load_gather(
                            rows, [kv[6], ridx, offs[6] + dh + lanes])
                        pooled[r, pl.ds(g * 32 + dh, 16)] = acc
                return 0

            lax.fori_loop(0, CB, accum, 0)

        pltpu.sync_copy(pooled,
                        out_hbm.at[pl.ds(base_row + sub_base, CB), :])
        return 0

    lax.fori_loop(0, NSUB, sub_chunk, 0)


@functools.partial(
    pl.kernel,
    out_type=jax.ShapeDtypeStruct((B, BASE_DIM), jnp.float32),
    mesh=plsc.VectorSubcoreMesh(core_axis_name="c", subcore_axis_name="s"),
    compiler_params=pltpu.CompilerParams(needs_layout_passes=False,
                                         use_tc_tiling_on_sc=False),
    scratch_types=[
        pltpu.VMEM((ROWS_PER_W * F,), jnp.int32),
        pltpu.VMEM((14, CB), jnp.int32),
        pltpu.VMEM((14, CB), jnp.int32),
        pltpu.VMEM((2, CB), jnp.float32),
        pltpu.VMEM((14, CB, 128), jnp.float32),
        pltpu.VMEM((CB, BASE_DIM), jnp.float32),
        pltpu.SemaphoreType.DMA,
    ],
)
def _sc_pool(x_hbm, t0, t1, t2, t3, out_hbm, xv, idxb, offb, wbuf, rows,
             pooled, sem):
    _sc_body(x_hbm, t0, t1, t2, t3, out_hbm, xv, idxb, offb, wbuf, rows,
             pooled, sem)


_BM = 1024


def _mm_body(p_ref, l0, l1, l2, l3, o_ref):
    w = jnp.concatenate([l0[...].T, l1[...].T, l2[...].T, l3[...].T], axis=0)
    o_ref[...] = jnp.dot(p_ref[...], w, preferred_element_type=jnp.float32)


def _project(pooled, l0, l1, l2, l3):
    wspec = pl.BlockSpec((BASE_DIM, BLOCK_DIM), lambda i: (0, 0))
    return pl.pallas_call(
        _mm_body,
        grid=(B // _BM,),
        in_specs=[pl.BlockSpec((_BM, BASE_DIM), lambda i: (i, 0)),
                  wspec, wspec, wspec, wspec],
        out_specs=pl.BlockSpec((_BM, BASE_DIM), lambda i: (i, 0)),
        out_shape=jax.ShapeDtypeStruct((B, BASE_DIM), jnp.float32),
    )(pooled, l0, l1, l2, l3)


def kernel(x, embed_w0, embed_w1, embed_w2, embed_w3,
           linear_w0, linear_w1, linear_w2, linear_w3):
    x_flat = x.astype(jnp.int32).reshape(-1)
    p0, p1, p2, p3 = _repack(embed_w0.T, embed_w1.T, embed_w2.T, embed_w3.T)
    pooled = _sc_pool(x_flat, p0, p1, p2, p3)
    return _project(pooled, linear_w0, linear_w1, linear_w2, linear_w3)
